# baseline (speedup 1.0000x reference)
import functools

import jax
import jax.numpy as jnp
from jax import lax
from jax.experimental import pallas as pl
from jax.experimental.pallas import tpu as pltpu
from jax.experimental.pallas import tpu_sc as plsc

N = 10000
NP = 10240
E = 320000
EP = 327680
H = 128
C = 16
N1 = 500
N1P = 512
E1 = 5000
E1P = 5120

NC = 2
NS = 16
NW = NC * NS

_BN_RS = 0.9999950000374997

_f32 = jnp.float32
_i32 = jnp.int32


def _mesh():
    return plsc.VectorSubcoreMesh(
        core_axis_name="c", subcore_axis_name="s", num_cores=NC, num_subcores=NS
    )


def _pre_body(dst0, p1, dst1, fidx1, deg_out, cnt_out, deg1_out, a_out,
              zbuf, cbuf, ones_a, ones_b, bufd0, bufp, bufd, fidx,
              deg_sh, cnt_sh, deg1_sh, a_sh,
              sem_a, sem_b, sem_c):
    cid = lax.axis_index("c")
    sid = lax.axis_index("s")
    w = cid * NS + sid

    z16 = jnp.zeros((16,), _f32)
    o16 = jnp.full((16,), 1.0, _f32)
    for k in range(2048 // 16):
        zbuf[pl.ds(k * 16, 16)] = z16
    for k in range(128 // 16):
        ones_a[pl.ds(k * 16, 16)] = o16
    for k in range(80 // 16):
        ones_b[pl.ds(k * 16, 16)] = o16

    pltpu.sync_copy(zbuf.at[pl.ds(0, 640)], deg_sh.at[pl.ds(sid * 640, 640)])
    pltpu.sync_copy(zbuf.at[pl.ds(0, 32)], cnt_sh.at[pl.ds(sid * 32, 32)])
    pltpu.sync_copy(zbuf.at[pl.ds(0, 32)], deg1_sh.at[pl.ds(sid * 32, 32)])

    @pl.loop(0, 8)
    def _(t):
        pltpu.sync_copy(zbuf, a_sh.at[pl.ds(sid * 16384 + t * 2048, 2048)])
    plsc.subcore_barrier()

    pltpu.sync_copy(dst0.at[pl.ds(w * 80, 80)], bufd0)

    @pl.loop(0, 80)
    def _(j):
        pltpu.async_copy(ones_a, deg_sh.at[bufd0.at[j]], sem_a, add=True)

    pltpu.sync_copy(p1.at[pl.ds(w * 4, 4)], bufp)
    for j in range(4):
        pltpu.async_copy(ones_b, cnt_sh.at[bufp.at[j]], sem_b, add=True)

    pltpu.sync_copy(dst1.at[pl.ds(sid * 4, 4)], bufd)
    for j in range(4):
        pltpu.async_copy(ones_b, deg1_sh.at[bufd.at[j]], sem_c, add=True)

    @pl.loop(0, 80)
    def _(j):
        pltpu.make_async_copy(ones_a, deg_sh.at[bufd0.at[j]], sem_a).wait()
    for j in range(4):
        pltpu.make_async_copy(ones_b, cnt_sh.at[bufp.at[j]], sem_b).wait()
        pltpu.make_async_copy(ones_b, deg1_sh.at[bufd.at[j]], sem_c).wait()
    plsc.subcore_barrier()

    row0 = cid * 32 + sid * 2
    pltpu.sync_copy(fidx1.at[pl.ds(row0, 2)], fidx)
    for j in range(2):
        pltpu.sync_copy(ones_b, a_sh.at[fidx.at[j]], add=True)
    plsc.subcore_barrier()

    pltpu.sync_copy(deg_sh.at[pl.ds(sid * 640, 640)], zbuf.at[pl.ds(0, 640)])
    pltpu.sync_copy(zbuf.at[pl.ds(0, 640)], deg_out.at[cid, sid])

    @pl.when(sid == 0)
    def _():
        pltpu.sync_copy(cnt_sh, cbuf)
        pltpu.sync_copy(cbuf, cnt_out.at[cid])

    @pl.when(sid == 1)
    def _():
        pltpu.sync_copy(deg1_sh, cbuf)
        pltpu.sync_copy(cbuf, deg1_out.at[cid])

    @pl.loop(0, 8)
    def _(t):
        pltpu.sync_copy(a_sh.at[pl.ds(sid * 16384 + t * 2048, 2048)], zbuf)
        pltpu.sync_copy(zbuf, a_out.at[cid, sid * 8 + t])


def _make_pre_kernel():
    return pl.kernel(
        _pre_body,
        out_type=(
            jax.ShapeDtypeStruct((NC, NS, 640), _f32),
            jax.ShapeDtypeStruct((NC, N1P), _f32),
            jax.ShapeDtypeStruct((NC, N1P), _f32),
            jax.ShapeDtypeStruct((NC, 128, 2048), _f32),
        ),
        mesh=_mesh(),
        scratch_types=[
            pltpu.VMEM((2048,), _f32),
            pltpu.VMEM((N1P,), _f32),
            pltpu.VMEM((128,), _f32),
            pltpu.VMEM((80,), _f32),
            pltpu.VMEM((80, 128), _i32),
            pltpu.VMEM((4, 80), _i32),
            pltpu.VMEM((4, 80), _i32),
            pltpu.VMEM((2, 80), _i32),
            pltpu.VMEM_SHARED((NP,), _f32),
            pltpu.VMEM_SHARED((N1P,), _f32),
            pltpu.VMEM_SHARED((N1P,), _f32),
            pltpu.VMEM_SHARED((N1P * N1P,), _f32),
            pltpu.SemaphoreType.DMA,
            pltpu.SemaphoreType.DMA,
            pltpu.SemaphoreType.DMA,
        ],
    )


def _make_row_scatter(n_in, n_out, ep, ch):
    eper = ep // NW
    nch = eper // ch
    assert eper % ch == 0 and ch % 8 == 0 and ch <= 128
    rps = n_out // NS
    S = 4 if (nch % 4 == 0 and nch >= 4) else 1

    def body(pk_r, tab_r, out_r, *refs):
        idxp = refs[0:S]
        rows = refs[S:2 * S]
        acc_sh = refs[2 * S]
        isem = refs[2 * S + 1:3 * S + 1]
        gsem = refs[3 * S + 1:4 * S + 1]
        ssem = refs[4 * S + 1:5 * S + 1]

        cid = lax.axis_index("c")
        sid = lax.axis_index("s")
        w = cid * NS + sid

        z16 = jnp.zeros((16,), _f32)

        @pl.loop(0, min(ch, rps))
        def _(r):
            for k in range(H // 16):
                rows[0][r, pl.ds(k * 16, 16)] = z16

        if rps >= ch:
            @pl.loop(0, rps // ch)
            def _(t):
                pltpu.sync_copy(rows[0],
                                acc_sh.at[pl.ds(sid * rps + t * ch, ch)])
        else:
            pltpu.sync_copy(rows[0].at[pl.ds(0, rps)],
                            acc_sh.at[pl.ds(sid * rps, rps)])
        plsc.subcore_barrier()

        def idx_start(i, b):
            pltpu.async_copy(pk_r.at[w * nch + i], idxp[b], isem[b])

        def idx_wait(b):
            pltpu.make_async_copy(pk_r.at[0], idxp[b], isem[b]).wait()

        def gather_start(i, b):
            pltpu.async_copy(tab_r.at[idxp[b].at[0]], rows[b], gsem[b])

        def gather_wait(b):
            pltpu.make_async_copy(tab_r.at[idxp[b].at[0]], rows[b],
                                  gsem[b]).wait()

        def scatter_start(b):
            pltpu.async_copy(rows[b], acc_sh.at[idxp[b].at[1]], ssem[b],
                             add=True)

        def scatter_wait(b):
            pltpu.make_async_copy(rows[b], acc_sh.at[idxp[b].at[1]],
                                  ssem[b]).wait()

        if S == 1:
            @pl.loop(0, nch)
            def _(i):
                idx_start(i, 0)
                idx_wait(0)
                pltpu.async_copy(tab_r.at[idxp[0].at[0]], rows[0],
                                 gsem[0]).wait()
                pltpu.sync_copy(rows[0], acc_sh.at[idxp[0].at[1]], add=True)
        else:
            idx_start(0, 0)
            idx_start(1, 1)
            idx_wait(0)
            gather_start(0, 0)

            def stage(i, s, first_peel=False, last_peel=False):
                if (not last_peel) or (s < S - 2):
                    if (not first_peel) or (s + 2 >= S):
                        if (not last_peel) or (nch - S + s + 2 >= S):
                            scatter_wait((s + 2) % S)
                    if not last_peel:
                        idx_start(i + 2, (s + 2) % S)
                    elif s < S - 2:
                        idx_start(i + 2, (s + 2) % S)
                if (not last_peel) or (s < S - 1):
                    idx_wait((s + 1) % S)
                    gather_start(i + 1, (s + 1) % S)
                gather_wait(s)
                scatter_start(s)

            if nch > S:
                for s in range(S):
                    stage(s, s, first_peel=True)

                @pl.loop(1, nch // S - 1)
                def _(o):
                    for s in range(S):
                        stage(o * S + s, s)

            for s in range(S):
                stage(nch - S + s, s, first_peel=(nch == S), last_peel=True)

            for b in range(S):
                scatter_wait(b)

        plsc.subcore_barrier()
        pltpu.sync_copy(acc_sh.at[pl.ds(sid * rps, rps)],
                        out_r.at[cid, pl.ds(sid * rps, rps)])

    return pl.kernel(
        body,
        out_type=jax.ShapeDtypeStruct((NC, n_out, H), _f32),
        mesh=_mesh(),
        scratch_types=(
            [pltpu.VMEM((2, ch), _i32) for _ in range(S)]
            + [pltpu.VMEM((ch, H), _f32) for _ in range(S)]
            + [pltpu.VMEM_SHARED((n_out, H), _f32)]
            + [pltpu.SemaphoreType.DMA for _ in range(3 * S)]
        ),
    )


_BR = 512


def _dinv_of(dref):
    deg = dref[0] + dref[1]
    return jnp.where(deg > 0, lax.rsqrt(deg), 0.0)


def _tc_in_body(x_ref, w_ref, d_ref, o_ref):
    dinv = _dinv_of(d_ref)
    o_ref[...] = jnp.dot(x_ref[...], w_ref[...],
                         preferred_element_type=_f32) * dinv[:, None]


def _tc_in(x, w, degp):
    return pl.pallas_call(
        _tc_in_body,
        grid=(NP // _BR,),
        in_specs=[
            pl.BlockSpec((_BR, H), lambda i: (i, 0)),
            pl.BlockSpec((H, H), lambda i: (0, 0)),
            pl.BlockSpec((NC, _BR), lambda i: (0, i)),
        ],
        out_specs=pl.BlockSpec((_BR, H), lambda i: (i, 0)),
        out_shape=jax.ShapeDtypeStruct((NP, H), _f32),
    )(x, w, degp)


def _tc_mid_body(a_ref, d_ref, w_ref, b_ref, g_ref, be_ref, o_ref):
    dinv = _dinv_of(d_ref)
    t = (a_ref[0] + a_ref[1]) * dinv[:, None] + b_ref[...]
    t = t * (g_ref[...] * _BN_RS) + be_ref[...]
    t = jnp.maximum(t, 0.0)
    o_ref[...] = jnp.dot(t, w_ref[...],
                         preferred_element_type=_f32) * dinv[:, None]


def _tc_mid(acc, degp, w, b, g, be):
    return pl.pallas_call(
        _tc_mid_body,
        grid=(NP // _BR,),
        in_specs=[
            pl.BlockSpec((NC, _BR, H), lambda i: (0, i, 0)),
            pl.BlockSpec((NC, _BR), lambda i: (0, i)),
            pl.BlockSpec((H, H), lambda i: (0, 0)),
            pl.BlockSpec((1, H), lambda i: (0, 0)),
            pl.BlockSpec((1, H), lambda i: (0, 0)),
            pl.BlockSpec((1, H), lambda i: (0, 0)),
        ],
        out_specs=pl.BlockSpec((_BR, H), lambda i: (i, 0)),
        out_shape=jax.ShapeDtypeStruct((NP, H), _f32),
    )(acc, degp, w, b, g, be)


def _tc_post_body(a_ref, d_ref, b_ref, g_ref, be_ref, o_ref):
    dinv = _dinv_of(d_ref)
    t = (a_ref[0] + a_ref[1]) * dinv[:, None] + b_ref[...]
    o_ref[...] = t * (g_ref[...] * _BN_RS) + be_ref[...]


def _tc_post(acc, degp, b, g, be):
    return pl.pallas_call(
        _tc_post_body,
        grid=(NP // _BR,),
        in_specs=[
            pl.BlockSpec((NC, _BR, H), lambda i: (0, i, 0)),
            pl.BlockSpec((NC, _BR), lambda i: (0, i)),
            pl.BlockSpec((1, H), lambda i: (0, 0)),
            pl.BlockSpec((1, H), lambda i: (0, 0)),
            pl.BlockSpec((1, H), lambda i: (0, 0)),
        ],
        out_specs=pl.BlockSpec((_BR, H), lambda i: (i, 0)),
        out_shape=jax.ShapeDtypeStruct((NP, H), _f32),
    )(acc, degp, b, g, be)


def _tc_tail_body(p_ref, c_ref, a_ref, d1_ref, wl_ref, bl_ref, xp_ref, wa_ref,
                  wb_ref, w1_ref, b0_ref, g0_ref, be0_ref, b1_ref, g1_ref,
                  be1_ref, wlf_ref, blf_ref, nl_ref, gl_ref):
    deg1 = d1_ref[0]
    dinv1 = jnp.where(deg1 > 0, lax.rsqrt(deg1), 0.0)
    adj = (a_ref[0] + a_ref[1]) * dinv1[:, None] * dinv1[None, :]
    cnt = c_ref[0] + c_ref[1]
    pooled = (p_ref[0] + p_ref[1]) / jnp.maximum(cnt, 1.0)[:, None]
    nl = jnp.dot(pooled, wl_ref[...], preferred_element_type=_f32) + bl_ref[...]
    nl_ref[...] = nl
    h1 = jnp.dot(nl, wa_ref[...], preferred_element_type=_f32)
    h1 = h1 + xp_ref[0][:, None] * wb_ref[...]
    t = jnp.dot(adj, h1, preferred_element_type=_f32) + b0_ref[...]
    t = t * (g0_ref[...] * _BN_RS) + be0_ref[...]
    t = jnp.maximum(t, 0.0)
    h2 = jnp.dot(t, w1_ref[...], preferred_element_type=_f32)
    t = jnp.dot(adj, h2, preferred_element_type=_f32) + b1_ref[...]
    t = t * (g1_ref[...] * _BN_RS) + be1_ref[...]
    rows = lax.broadcasted_iota(_i32, (N1P, H), 0)
    t = jnp.where(rows < N1, t, 0.0)
    g_mean = jnp.sum(t, axis=0, keepdims=True) * (1.0 / N1)
    gl_ref[...] = jnp.dot(g_mean, wlf_ref[...],
                          preferred_element_type=_f32) + blf_ref[...]


def _tc_tail(pacc, cntp, adjp, deg1p, wl, bl, xp, wa, wb, w1, b0, g0, be0,
             b1, g1, be1, wlf, blf):
    return pl.pallas_call(
        _tc_tail_body,
        out_shape=(
            jax.ShapeDtypeStruct((N1P, H), _f32),
            jax.ShapeDtypeStruct((1, H), _f32),
        ),
    )(pacc, cntp, adjp, deg1p, wl, bl, xp, wa, wb, w1, b0, g0, be0, b1, g1,
      be1, wlf, blf)


def kernel(x, edge_index, pool1, x_pool1, edge_index_pool1,
           W0_p0, b0_p0, g0_p0, be0_p0, W1_p0, b1_p0, g1_p0, be1_p0,
           Wl_p0, bl_p0,
           W0_p1, b0_p1, g0_p1, be0_p1, W1_p1, b1_p1, g1_p1, be1_p1,
           Wl_p1, bl_p1):
    ei = edge_index.astype(_i32)
    pad_big = N + (jnp.arange(EP - E, dtype=_i32) % (NP - N))
    src_p = jnp.concatenate([ei[0], pad_big])
    dst_p = jnp.concatenate([ei[1], pad_big])
    x_p = jnp.pad(x, ((0, NP - N), (0, 0)))

    pool1_p = jnp.concatenate(
        [pool1.astype(_i32), jnp.full((NP - N,), N1, _i32)])
    ei1 = edge_index_pool1.astype(_i32)
    pad_sm = N1 + (jnp.arange(E1P - E1, dtype=_i32) % (N1P - N1))
    src1_p = jnp.concatenate([ei1[0], pad_sm])
    dst1_p = jnp.concatenate([ei1[1], pad_sm])

    dst_2d = dst_p.reshape(EP // 128, 128)
    pool1_2d = pool1_p.reshape(NP // 80, 80)
    dst1_2d = dst1_p.reshape(E1P // 80, 80)
    fidx1 = (dst1_p * N1P + src1_p).reshape(E1P // 80, 80)
    pk_big = jnp.stack([src_p.reshape(NW, 128, 80),
                        dst_p.reshape(NW, 128, 80)], axis=2)
    pk_big = pk_big.reshape(NW * 128, 2, 80)

    iota_np = jnp.arange(NP, dtype=_i32)
    pk_pool = jnp.stack([iota_np.reshape(NW, 4, 80),
                         pool1_p.reshape(NW, 4, 80)], axis=2)
    pk_pool = pk_pool.reshape(NW * 4, 2, 80)
    xp_row = jnp.pad(x_pool1, ((0, N1P - N1), (0, 0))).reshape(1, N1P)

    wl0 = jnp.pad(Wl_p0, ((0, 0), (0, H - C)))
    bl0 = jnp.pad(bl_p0, (0, H - C)).reshape(1, H)
    wa = jnp.pad(W0_p1[:C], ((0, H - C), (0, 0)))
    wb = W0_p1[C:C + 1]
    wl1 = jnp.pad(Wl_p1, ((0, 0), (0, H - C)))
    bl1 = jnp.pad(bl_p1, (0, H - C)).reshape(1, H)

    r = lambda v: v.reshape(1, H)

    pre_k = _make_pre_kernel()
    scat_big = _make_row_scatter(NP, NP, EP, 80)
    scat_pool = _make_row_scatter(NP, N1P, NP, 80)

    degp, cntp, deg1p, adjp = pre_k(dst_2d, pool1_2d, dst1_2d, fidx1)
    degp = degp.reshape(NC, NP)
    adjp = adjp.reshape(NC, N1P, N1P)

    hs0 = _tc_in(x_p, W0_p0, degp)
    acc0 = scat_big(pk_big, hs0)
    hs1 = _tc_mid(acc0, degp, W1_p0, r(b0_p0), r(g0_p0), r(be0_p0))
    acc1 = scat_big(pk_big, hs1)
    hf = _tc_post(acc1, degp, r(b1_p0), r(g1_p0), r(be1_p0))

    pacc = scat_pool(pk_pool, hf)
    nlf, gl = _tc_tail(pacc, cntp, adjp, deg1p, wl0, bl0, xp_row, wa, wb,
                       W1_p1, r(b0_p1), r(g0_p1), r(be0_p1),
                       r(b1_p1), r(g1_p1), r(be1_p1), wl1, bl1)

    return gl[:, :C], nlf[:N1, :C]


# speedup vs baseline: 1.0667x; 1.0667x over previous
import functools

import jax
import jax.numpy as jnp
from jax import lax
from jax.experimental import pallas as pl
from jax.experimental.pallas import tpu as pltpu
from jax.experimental.pallas import tpu_sc as plsc

N = 10000
NP = 10240
E = 320000
EP = 327680
H = 128
C = 16
N1 = 500
N1P = 512
E1 = 5000
E1P = 5120

NC = 2
NS = 16
NW = NC * NS

_BN_RS = 0.9999950000374997

_f32 = jnp.float32
_i32 = jnp.int32


def _mesh():
    return plsc.VectorSubcoreMesh(
        core_axis_name="c", subcore_axis_name="s", num_cores=NC, num_subcores=NS
    )


def _pre_body(dst0, p1, dst1, fidx1, deg_out, cnt_out, deg1_out, a_out,
              zbuf, cbuf, ones_a, ones_b, bufd0, bufp, bufd, fidx,
              deg_sh, cnt_sh, deg1_sh, a_sh,
              sem_a, sem_b, sem_c):
    cid = lax.axis_index("c")
    sid = lax.axis_index("s")
    w = cid * NS + sid

    z16 = jnp.zeros((16,), _f32)
    o16 = jnp.full((16,), 1.0, _f32)
    for k in range(2048 // 16):
        zbuf[pl.ds(k * 16, 16)] = z16
    for k in range(128 // 16):
        ones_a[pl.ds(k * 16, 16)] = o16
    for k in range(80 // 16):
        ones_b[pl.ds(k * 16, 16)] = o16

    pltpu.sync_copy(zbuf.at[pl.ds(0, 640)], deg_sh.at[pl.ds(sid * 640, 640)])
    pltpu.sync_copy(zbuf.at[pl.ds(0, 32)], cnt_sh.at[pl.ds(sid * 32, 32)])
    pltpu.sync_copy(zbuf.at[pl.ds(0, 32)], deg1_sh.at[pl.ds(sid * 32, 32)])

    @pl.loop(0, 8)
    def _(t):
        pltpu.sync_copy(zbuf, a_sh.at[pl.ds(sid * 16384 + t * 2048, 2048)])
    plsc.subcore_barrier()

    pltpu.sync_copy(dst0.at[pl.ds(w * 80, 80)], bufd0)

    @pl.loop(0, 80)
    def _(j):
        pltpu.async_copy(ones_a, deg_sh.at[bufd0.at[j]], sem_a, add=True)

    pltpu.sync_copy(p1.at[pl.ds(w * 4, 4)], bufp)
    for j in range(4):
        pltpu.async_copy(ones_b, cnt_sh.at[bufp.at[j]], sem_b, add=True)

    pltpu.sync_copy(dst1.at[pl.ds(sid * 4, 4)], bufd)
    for j in range(4):
        pltpu.async_copy(ones_b, deg1_sh.at[bufd.at[j]], sem_c, add=True)

    @pl.loop(0, 80)
    def _(j):
        pltpu.make_async_copy(ones_a, deg_sh.at[bufd0.at[j]], sem_a).wait()
    for j in range(4):
        pltpu.make_async_copy(ones_b, cnt_sh.at[bufp.at[j]], sem_b).wait()
        pltpu.make_async_copy(ones_b, deg1_sh.at[bufd.at[j]], sem_c).wait()
    plsc.subcore_barrier()

    row0 = cid * 32 + sid * 2
    pltpu.sync_copy(fidx1.at[pl.ds(row0, 2)], fidx)
    for j in range(2):
        pltpu.sync_copy(ones_b, a_sh.at[fidx.at[j]], add=True)
    plsc.subcore_barrier()

    pltpu.sync_copy(deg_sh.at[pl.ds(sid * 640, 640)], zbuf.at[pl.ds(0, 640)])
    pltpu.sync_copy(zbuf.at[pl.ds(0, 640)], deg_out.at[cid, sid])

    @pl.when(sid == 0)
    def _():
        pltpu.sync_copy(cnt_sh, cbuf)
        pltpu.sync_copy(cbuf, cnt_out.at[cid])

    @pl.when(sid == 1)
    def _():
        pltpu.sync_copy(deg1_sh, cbuf)
        pltpu.sync_copy(cbuf, deg1_out.at[cid])

    @pl.loop(0, 8)
    def _(t):
        pltpu.sync_copy(a_sh.at[pl.ds(sid * 16384 + t * 2048, 2048)], zbuf)
        pltpu.sync_copy(zbuf, a_out.at[cid, sid * 8 + t])


def _make_pre_kernel():
    return pl.kernel(
        _pre_body,
        out_type=(
            jax.ShapeDtypeStruct((NC, NS, 640), _f32),
            jax.ShapeDtypeStruct((NC, N1P), _f32),
            jax.ShapeDtypeStruct((NC, N1P), _f32),
            jax.ShapeDtypeStruct((NC, 128, 2048), _f32),
        ),
        mesh=_mesh(),
        scratch_types=[
            pltpu.VMEM((2048,), _f32),
            pltpu.VMEM((N1P,), _f32),
            pltpu.VMEM((128,), _f32),
            pltpu.VMEM((80,), _f32),
            pltpu.VMEM((80, 128), _i32),
            pltpu.VMEM((4, 80), _i32),
            pltpu.VMEM((4, 80), _i32),
            pltpu.VMEM((2, 80), _i32),
            pltpu.VMEM_SHARED((NP,), _f32),
            pltpu.VMEM_SHARED((N1P,), _f32),
            pltpu.VMEM_SHARED((N1P,), _f32),
            pltpu.VMEM_SHARED((N1P * N1P,), _f32),
            pltpu.SemaphoreType.DMA,
            pltpu.SemaphoreType.DMA,
            pltpu.SemaphoreType.DMA,
        ],
    )


def _make_row_scatter(n_in, n_out, ep, ch):
    eper = ep // NW
    nch = eper // ch
    assert eper % ch == 0 and ch % 8 == 0 and ch <= 128
    rps = n_out // NS
    S = 4 if (nch % 4 == 0 and nch >= 4) else 1

    def body(src_r, dst_r, tab_r, out_r, *refs):
        idxs = refs[0:S]
        idxd = refs[S:2 * S]
        rows = refs[2 * S:3 * S]
        acc_sh = refs[3 * S]
        isem = refs[3 * S + 1:4 * S + 1]
        gsem = refs[4 * S + 1:5 * S + 1]
        ssem = refs[5 * S + 1:6 * S + 1]

        cid = lax.axis_index("c")
        sid = lax.axis_index("s")
        w = cid * NS + sid

        z16 = jnp.zeros((16,), _f32)

        @pl.loop(0, min(ch, rps))
        def _(r):
            for k in range(H // 16):
                rows[0][r, pl.ds(k * 16, 16)] = z16

        if rps >= ch:
            @pl.loop(0, rps // ch)
            def _(t):
                pltpu.sync_copy(rows[0],
                                acc_sh.at[pl.ds(sid * rps + t * ch, ch)])
        else:
            pltpu.sync_copy(rows[0].at[pl.ds(0, rps)],
                            acc_sh.at[pl.ds(sid * rps, rps)])
        plsc.subcore_barrier()

        def idx_start(i, b):
            base = w * eper + i * ch
            pltpu.async_copy(src_r.at[pl.ds(base, ch)], idxs[b], isem[b])
            pltpu.async_copy(dst_r.at[pl.ds(base, ch)], idxd[b], isem[b])

        def idx_wait(b):
            pltpu.make_async_copy(src_r.at[pl.ds(0, ch)], idxs[b],
                                  isem[b]).wait()
            pltpu.make_async_copy(dst_r.at[pl.ds(0, ch)], idxd[b],
                                  isem[b]).wait()

        def gather_start(i, b):
            pltpu.async_copy(tab_r.at[idxs[b]], rows[b], gsem[b])

        def gather_wait(b):
            pltpu.make_async_copy(tab_r.at[idxs[b]], rows[b], gsem[b]).wait()

        def scatter_start(b):
            pltpu.async_copy(rows[b], acc_sh.at[idxd[b]], ssem[b], add=True)

        def scatter_wait(b):
            pltpu.make_async_copy(rows[b], acc_sh.at[idxd[b]],
                                  ssem[b]).wait()

        if S == 1:
            @pl.loop(0, nch)
            def _(i):
                idx_start(i, 0)
                idx_wait(0)
                pltpu.async_copy(tab_r.at[idxs[0]], rows[0], gsem[0]).wait()
                pltpu.sync_copy(rows[0], acc_sh.at[idxd[0]], add=True)
        else:
            idx_start(0, 0)
            idx_start(1, 1)
            idx_wait(0)
            gather_start(0, 0)

            def stage(i, s, first_peel=False, last_peel=False):
                if (not last_peel) or (s < S - 2):
                    if (not first_peel) or (s + 2 >= S):
                        if (not last_peel) or (nch - S + s + 2 >= S):
                            scatter_wait((s + 2) % S)
                    if not last_peel:
                        idx_start(i + 2, (s + 2) % S)
                    elif s < S - 2:
                        idx_start(i + 2, (s + 2) % S)
                if (not last_peel) or (s < S - 1):
                    idx_wait((s + 1) % S)
                    gather_start(i + 1, (s + 1) % S)
                gather_wait(s)
                scatter_start(s)

            if nch > S:
                for s in range(S):
                    stage(s, s, first_peel=True)

                @pl.loop(1, nch // S - 1)
                def _(o):
                    for s in range(S):
                        stage(o * S + s, s)

            for s in range(S):
                stage(nch - S + s, s, first_peel=(nch == S), last_peel=True)

            for b in range(S):
                scatter_wait(b)

        plsc.subcore_barrier()
        pltpu.sync_copy(acc_sh.at[pl.ds(sid * rps, rps)],
                        out_r.at[cid, pl.ds(sid * rps, rps)])

    return pl.kernel(
        body,
        out_type=jax.ShapeDtypeStruct((NC, n_out, H), _f32),
        mesh=_mesh(),
        scratch_types=(
            [pltpu.VMEM((ch,), _i32) for _ in range(S)]
            + [pltpu.VMEM((ch,), _i32) for _ in range(S)]
            + [pltpu.VMEM((ch, H), _f32) for _ in range(S)]
            + [pltpu.VMEM_SHARED((n_out, H), _f32)]
            + [pltpu.SemaphoreType.DMA for _ in range(3 * S)]
        ),
    )


_BR = 1024


def _dinv_of(dref):
    deg = dref[0] + dref[1]
    return jnp.where(deg > 0, lax.rsqrt(deg), 0.0)


def _tc_in_body(x_ref, w_ref, d_ref, o_ref):
    dinv = _dinv_of(d_ref)
    o_ref[...] = jnp.dot(x_ref[...], w_ref[...],
                         preferred_element_type=_f32) * dinv[:, None]


def _tc_in(x, w, degp):
    return pl.pallas_call(
        _tc_in_body,
        grid=(NP // _BR,),
        in_specs=[
            pl.BlockSpec((_BR, H), lambda i: (i, 0)),
            pl.BlockSpec((H, H), lambda i: (0, 0)),
            pl.BlockSpec((NC, _BR), lambda i: (0, i)),
        ],
        out_specs=pl.BlockSpec((_BR, H), lambda i: (i, 0)),
        out_shape=jax.ShapeDtypeStruct((NP, H), _f32),
    )(x, w, degp)


def _tc_mid_body(a_ref, d_ref, w_ref, b_ref, g_ref, be_ref, o_ref):
    dinv = _dinv_of(d_ref)
    t = (a_ref[0] + a_ref[1]) * dinv[:, None] + b_ref[...]
    t = t * (g_ref[...] * _BN_RS) + be_ref[...]
    t = jnp.maximum(t, 0.0)
    o_ref[...] = jnp.dot(t, w_ref[...],
                         preferred_element_type=_f32) * dinv[:, None]


def _tc_mid(acc, degp, w, b, g, be):
    return pl.pallas_call(
        _tc_mid_body,
        grid=(NP // _BR,),
        in_specs=[
            pl.BlockSpec((NC, _BR, H), lambda i: (0, i, 0)),
            pl.BlockSpec((NC, _BR), lambda i: (0, i)),
            pl.BlockSpec((H, H), lambda i: (0, 0)),
            pl.BlockSpec((1, H), lambda i: (0, 0)),
            pl.BlockSpec((1, H), lambda i: (0, 0)),
            pl.BlockSpec((1, H), lambda i: (0, 0)),
        ],
        out_specs=pl.BlockSpec((_BR, H), lambda i: (i, 0)),
        out_shape=jax.ShapeDtypeStruct((NP, H), _f32),
    )(acc, degp, w, b, g, be)


def _tc_post_body(a_ref, d_ref, b_ref, g_ref, be_ref, o_ref):
    dinv = _dinv_of(d_ref)
    t = (a_ref[0] + a_ref[1]) * dinv[:, None] + b_ref[...]
    o_ref[...] = t * (g_ref[...] * _BN_RS) + be_ref[...]


def _tc_post(acc, degp, b, g, be):
    return pl.pallas_call(
        _tc_post_body,
        grid=(NP // _BR,),
        in_specs=[
            pl.BlockSpec((NC, _BR, H), lambda i: (0, i, 0)),
            pl.BlockSpec((NC, _BR), lambda i: (0, i)),
            pl.BlockSpec((1, H), lambda i: (0, 0)),
            pl.BlockSpec((1, H), lambda i: (0, 0)),
            pl.BlockSpec((1, H), lambda i: (0, 0)),
        ],
        out_specs=pl.BlockSpec((_BR, H), lambda i: (i, 0)),
        out_shape=jax.ShapeDtypeStruct((NP, H), _f32),
    )(acc, degp, b, g, be)


def _tc_tail_body(p_ref, c_ref, a_ref, d1_ref, wl_ref, bl_ref, xp_ref, wa_ref,
                  wb_ref, w1_ref, b0_ref, g0_ref, be0_ref, b1_ref, g1_ref,
                  be1_ref, wlf_ref, blf_ref, nl_ref, gl_ref):
    deg1 = d1_ref[0]
    dinv1 = jnp.where(deg1 > 0, lax.rsqrt(deg1), 0.0)
    adj = (a_ref[0] + a_ref[1]) * dinv1[:, None] * dinv1[None, :]
    cnt = c_ref[0] + c_ref[1]
    pooled = (p_ref[0] + p_ref[1]) / jnp.maximum(cnt, 1.0)[:, None]
    nl = jnp.dot(pooled, wl_ref[...], preferred_element_type=_f32) + bl_ref[...]
    nl_ref[...] = nl
    h1 = jnp.dot(nl, wa_ref[...], preferred_element_type=_f32)
    h1 = h1 + xp_ref[0][:, None] * wb_ref[...]
    t = jnp.dot(adj, h1, preferred_element_type=_f32) + b0_ref[...]
    t = t * (g0_ref[...] * _BN_RS) + be0_ref[...]
    t = jnp.maximum(t, 0.0)
    h2 = jnp.dot(t, w1_ref[...], preferred_element_type=_f32)
    t = jnp.dot(adj, h2, preferred_element_type=_f32) + b1_ref[...]
    t = t * (g1_ref[...] * _BN_RS) + be1_ref[...]
    rows = lax.broadcasted_iota(_i32, (N1P, H), 0)
    t = jnp.where(rows < N1, t, 0.0)
    g_mean = jnp.sum(t, axis=0, keepdims=True) * (1.0 / N1)
    gl_ref[...] = jnp.dot(g_mean, wlf_ref[...],
                          preferred_element_type=_f32) + blf_ref[...]


def _tc_tail(pacc, cntp, adjp, deg1p, wl, bl, xp, wa, wb, w1, b0, g0, be0,
             b1, g1, be1, wlf, blf):
    return pl.pallas_call(
        _tc_tail_body,
        out_shape=(
            jax.ShapeDtypeStruct((N1P, H), _f32),
            jax.ShapeDtypeStruct((1, H), _f32),
        ),
    )(pacc, cntp, adjp, deg1p, wl, bl, xp, wa, wb, w1, b0, g0, be0, b1, g1,
      be1, wlf, blf)


def kernel(x, edge_index, pool1, x_pool1, edge_index_pool1,
           W0_p0, b0_p0, g0_p0, be0_p0, W1_p0, b1_p0, g1_p0, be1_p0,
           Wl_p0, bl_p0,
           W0_p1, b0_p1, g0_p1, be0_p1, W1_p1, b1_p1, g1_p1, be1_p1,
           Wl_p1, bl_p1):
    ei = edge_index.astype(_i32)
    pad_big = N + (jnp.arange(EP - E, dtype=_i32) % (NP - N))
    src_p = jnp.concatenate([ei[0], pad_big])
    dst_p = jnp.concatenate([ei[1], pad_big])
    x_p = jnp.pad(x, ((0, NP - N), (0, 0)))

    pool1_p = jnp.concatenate(
        [pool1.astype(_i32), jnp.full((NP - N,), N1, _i32)])
    ei1 = edge_index_pool1.astype(_i32)
    pad_sm = N1 + (jnp.arange(E1P - E1, dtype=_i32) % (N1P - N1))
    src1_p = jnp.concatenate([ei1[0], pad_sm])
    dst1_p = jnp.concatenate([ei1[1], pad_sm])

    dst_2d = dst_p.reshape(EP // 128, 128)
    pool1_2d = pool1_p.reshape(NP // 80, 80)
    dst1_2d = dst1_p.reshape(E1P // 80, 80)
    fidx1 = (dst1_p * N1P + src1_p).reshape(E1P // 80, 80)

    iota_np = jnp.arange(NP, dtype=_i32)
    xp_row = jnp.pad(x_pool1, ((0, N1P - N1), (0, 0))).reshape(1, N1P)

    wl0 = jnp.pad(Wl_p0, ((0, 0), (0, H - C)))
    bl0 = jnp.pad(bl_p0, (0, H - C)).reshape(1, H)
    wa = jnp.pad(W0_p1[:C], ((0, H - C), (0, 0)))
    wb = W0_p1[C:C + 1]
    wl1 = jnp.pad(Wl_p1, ((0, 0), (0, H - C)))
    bl1 = jnp.pad(bl_p1, (0, H - C)).reshape(1, H)

    r = lambda v: v.reshape(1, H)

    pre_k = _make_pre_kernel()
    scat_big = _make_row_scatter(NP, NP, EP, 80)
    scat_pool = _make_row_scatter(NP, N1P, NP, 80)

    degp, cntp, deg1p, adjp = pre_k(dst_2d, pool1_2d, dst1_2d, fidx1)
    degp = degp.reshape(NC, NP)
    adjp = adjp.reshape(NC, N1P, N1P)

    hs0 = _tc_in(x_p, W0_p0, degp)
    acc0 = scat_big(src_p, dst_p, hs0)
    hs1 = _tc_mid(acc0, degp, W1_p0, r(b0_p0), r(g0_p0), r(be0_p0))
    acc1 = scat_big(src_p, dst_p, hs1)
    hf = _tc_post(acc1, degp, r(b1_p0), r(g1_p0), r(be1_p0))

    pacc = scat_pool(iota_np, pool1_p, hf)
    nlf, gl = _tc_tail(pacc, cntp, adjp, deg1p, wl0, bl0, xp_row, wa, wb,
                       W1_p1, r(b0_p1), r(g0_p1), r(be0_p1),
                       r(b1_p1), r(g1_p1), r(be1_p1), wl1, bl1)

    return gl[:, :C], nlf[:N1, :C]


# speedup vs baseline: 1.0912x; 1.0230x over previous
import functools

import jax
import jax.numpy as jnp
from jax import lax
from jax.experimental import pallas as pl
from jax.experimental.pallas import tpu as pltpu
from jax.experimental.pallas import tpu_sc as plsc

N = 10000
NP = 10240
E = 320000
EP = 327680
H = 128
C = 16
N1 = 500
N1P = 512
E1 = 5000
E1P = 5120

NC = 2
NS = 16
NW = NC * NS

_BN_RS = 0.9999950000374997

_f32 = jnp.float32
_i32 = jnp.int32


def _mesh():
    return plsc.VectorSubcoreMesh(
        core_axis_name="c", subcore_axis_name="s", num_cores=NC, num_subcores=NS
    )


def _pre_body(dst0, p1, dst1, fidx1, deg_out, cnt_out, deg1_out, a_out,
              zbuf, cbuf, ones_a, ones_b, bufd0, bufp, bufd, fidx,
              deg_sh, cnt_sh, deg1_sh, a_sh,
              sem_a, sem_b, sem_c):
    cid = lax.axis_index("c")
    sid = lax.axis_index("s")
    w = cid * NS + sid

    z16 = jnp.zeros((16,), _f32)
    o16 = jnp.full((16,), 1.0, _f32)
    for k in range(2048 // 16):
        zbuf[pl.ds(k * 16, 16)] = z16
    for k in range(128 // 16):
        ones_a[pl.ds(k * 16, 16)] = o16
    for k in range(80 // 16):
        ones_b[pl.ds(k * 16, 16)] = o16

    pltpu.sync_copy(zbuf.at[pl.ds(0, 640)], deg_sh.at[pl.ds(sid * 640, 640)])
    pltpu.sync_copy(zbuf.at[pl.ds(0, 32)], cnt_sh.at[pl.ds(sid * 32, 32)])
    pltpu.sync_copy(zbuf.at[pl.ds(0, 32)], deg1_sh.at[pl.ds(sid * 32, 32)])

    @pl.loop(0, 8)
    def _(t):
        pltpu.sync_copy(zbuf, a_sh.at[pl.ds(sid * 16384 + t * 2048, 2048)])
    plsc.subcore_barrier()

    pltpu.sync_copy(dst0.at[pl.ds(w * 80, 80)], bufd0)

    @pl.loop(0, 80)
    def _(j):
        pltpu.async_copy(ones_a, deg_sh.at[bufd0.at[j]], sem_a, add=True)

    pltpu.sync_copy(p1.at[pl.ds(w * 4, 4)], bufp)
    for j in range(4):
        pltpu.async_copy(ones_b, cnt_sh.at[bufp.at[j]], sem_b, add=True)

    pltpu.sync_copy(dst1.at[pl.ds(sid * 4, 4)], bufd)
    for j in range(4):
        pltpu.async_copy(ones_b, deg1_sh.at[bufd.at[j]], sem_c, add=True)

    @pl.loop(0, 80)
    def _(j):
        pltpu.make_async_copy(ones_a, deg_sh.at[bufd0.at[j]], sem_a).wait()
    for j in range(4):
        pltpu.make_async_copy(ones_b, cnt_sh.at[bufp.at[j]], sem_b).wait()
        pltpu.make_async_copy(ones_b, deg1_sh.at[bufd.at[j]], sem_c).wait()
    plsc.subcore_barrier()

    row0 = cid * 32 + sid * 2
    pltpu.sync_copy(fidx1.at[pl.ds(row0, 2)], fidx)
    for j in range(2):
        pltpu.sync_copy(ones_b, a_sh.at[fidx.at[j]], add=True)
    plsc.subcore_barrier()

    pltpu.sync_copy(deg_sh.at[pl.ds(sid * 640, 640)], zbuf.at[pl.ds(0, 640)])
    pltpu.sync_copy(zbuf.at[pl.ds(0, 640)], deg_out.at[cid, sid])

    @pl.when(sid == 0)
    def _():
        pltpu.sync_copy(cnt_sh, cbuf)
        pltpu.sync_copy(cbuf, cnt_out.at[cid])

    @pl.when(sid == 1)
    def _():
        pltpu.sync_copy(deg1_sh, cbuf)
        pltpu.sync_copy(cbuf, deg1_out.at[cid])

    @pl.loop(0, 8)
    def _(t):
        pltpu.sync_copy(a_sh.at[pl.ds(sid * 16384 + t * 2048, 2048)], zbuf)
        pltpu.sync_copy(zbuf, a_out.at[cid, sid * 8 + t])


def _make_pre_kernel():
    return pl.kernel(
        _pre_body,
        out_type=(
            jax.ShapeDtypeStruct((NC, NS, 640), _f32),
            jax.ShapeDtypeStruct((NC, N1P), _f32),
            jax.ShapeDtypeStruct((NC, N1P), _f32),
            jax.ShapeDtypeStruct((NC, 128, 2048), _f32),
        ),
        mesh=_mesh(),
        scratch_types=[
            pltpu.VMEM((2048,), _f32),
            pltpu.VMEM((N1P,), _f32),
            pltpu.VMEM((128,), _f32),
            pltpu.VMEM((80,), _f32),
            pltpu.VMEM((80, 128), _i32),
            pltpu.VMEM((4, 80), _i32),
            pltpu.VMEM((4, 80), _i32),
            pltpu.VMEM((2, 80), _i32),
            pltpu.VMEM_SHARED((NP,), _f32),
            pltpu.VMEM_SHARED((N1P,), _f32),
            pltpu.VMEM_SHARED((N1P,), _f32),
            pltpu.VMEM_SHARED((N1P * N1P,), _f32),
            pltpu.SemaphoreType.DMA,
            pltpu.SemaphoreType.DMA,
            pltpu.SemaphoreType.DMA,
        ],
    )


def _make_row_scatter(n_in, n_out, ep, ch):
    eper = ep // NW
    nch = eper // ch
    assert eper % ch == 0 and ch % 8 == 0 and ch <= 128
    rps = n_out // NS
    S = 4 if (nch % 4 == 0 and nch >= 4) else 1

    def body(src_r, dst_r, tab_r, out_r, *refs):
        idxs = refs[0:S]
        idxd = refs[S:2 * S]
        rows = refs[2 * S:3 * S]
        acc_sh = refs[3 * S]
        isem = refs[3 * S + 1:4 * S + 1]
        gsem = refs[4 * S + 1:5 * S + 1]
        ssem = refs[5 * S + 1:6 * S + 1]

        cid = lax.axis_index("c")
        sid = lax.axis_index("s")
        w = cid * NS + sid

        z16 = jnp.zeros((16,), _f32)

        @pl.loop(0, min(ch, rps))
        def _(r):
            for k in range(H // 16):
                rows[0][r, pl.ds(k * 16, 16)] = z16

        if rps >= ch:
            @pl.loop(0, rps // ch)
            def _(t):
                pltpu.sync_copy(rows[0],
                                acc_sh.at[pl.ds(sid * rps + t * ch, ch)])
        else:
            pltpu.sync_copy(rows[0].at[pl.ds(0, rps)],
                            acc_sh.at[pl.ds(sid * rps, rps)])
        plsc.subcore_barrier()

        def idx_start(i, b):
            base = w * eper + i * ch
            pltpu.async_copy(src_r.at[pl.ds(base, ch)], idxs[b], isem[b])
            pltpu.async_copy(dst_r.at[pl.ds(base, ch)], idxd[b], isem[b])

        def idx_wait(b):
            pltpu.make_async_copy(src_r.at[pl.ds(0, ch)], idxs[b],
                                  isem[b]).wait()
            pltpu.make_async_copy(dst_r.at[pl.ds(0, ch)], idxd[b],
                                  isem[b]).wait()

        def gather_start(i, b):
            pltpu.async_copy(tab_r.at[idxs[b]], rows[b], gsem[b])

        def gather_wait(b):
            pltpu.make_async_copy(tab_r.at[idxs[b]], rows[b], gsem[b]).wait()

        def scatter_start(b):
            pltpu.async_copy(rows[b], acc_sh.at[idxd[b]], ssem[b], add=True)

        def scatter_wait(b):
            pltpu.make_async_copy(rows[b], acc_sh.at[idxd[b]],
                                  ssem[b]).wait()

        if S == 1:
            @pl.loop(0, nch)
            def _(i):
                idx_start(i, 0)
                idx_wait(0)
                pltpu.async_copy(tab_r.at[idxs[0]], rows[0], gsem[0]).wait()
                pltpu.sync_copy(rows[0], acc_sh.at[idxd[0]], add=True)
        else:
            idx_start(0, 0)
            idx_start(1, 1)
            idx_wait(0)
            gather_start(0, 0)

            def stage(i, s, first_peel=False, last_peel=False):
                if (not last_peel) or (s < S - 2):
                    if (not first_peel) or (s + 2 >= S):
                        if (not last_peel) or (nch - S + s + 2 >= S):
                            scatter_wait((s + 2) % S)
                    if not last_peel:
                        idx_start(i + 2, (s + 2) % S)
                    elif s < S - 2:
                        idx_start(i + 2, (s + 2) % S)
                if (not last_peel) or (s < S - 1):
                    idx_wait((s + 1) % S)
                    gather_start(i + 1, (s + 1) % S)
                gather_wait(s)
                scatter_start(s)

            if nch > S:
                for s in range(S):
                    stage(s, s, first_peel=True)

                @pl.loop(1, nch // S - 1)
                def _(o):
                    for s in range(S):
                        stage(o * S + s, s)

            for s in range(S):
                stage(nch - S + s, s, first_peel=(nch == S), last_peel=True)

            for b in range(S):
                scatter_wait(b)

        plsc.subcore_barrier()
        pltpu.sync_copy(acc_sh.at[pl.ds(sid * rps, rps)],
                        out_r.at[cid, pl.ds(sid * rps, rps)])

    return pl.kernel(
        body,
        out_type=jax.ShapeDtypeStruct((NC, n_out, H), _f32),
        mesh=_mesh(),
        scratch_types=(
            [pltpu.VMEM((ch,), _i32) for _ in range(S)]
            + [pltpu.VMEM((ch,), _i32) for _ in range(S)]
            + [pltpu.VMEM((ch, H), _f32) for _ in range(S)]
            + [pltpu.VMEM_SHARED((n_out, H), _f32)]
            + [pltpu.SemaphoreType.DMA for _ in range(3 * S)]
        ),
    )


_BR = 2048


def _dinv_of(dref):
    deg = dref[0] + dref[1]
    return jnp.where(deg > 0, lax.rsqrt(deg), 0.0)


def _tc_in_body(x_ref, w_ref, d_ref, o_ref):
    dinv = _dinv_of(d_ref)
    o_ref[...] = jnp.dot(x_ref[...], w_ref[...],
                         preferred_element_type=_f32) * dinv[:, None]


def _tc_in(x, w, degp):
    return pl.pallas_call(
        _tc_in_body,
        grid=(NP // _BR,),
        in_specs=[
            pl.BlockSpec((_BR, H), lambda i: (i, 0)),
            pl.BlockSpec((H, H), lambda i: (0, 0)),
            pl.BlockSpec((NC, _BR), lambda i: (0, i)),
        ],
        out_specs=pl.BlockSpec((_BR, H), lambda i: (i, 0)),
        out_shape=jax.ShapeDtypeStruct((NP, H), _f32),
    )(x, w, degp)


def _tc_mid_body(a_ref, d_ref, w_ref, b_ref, g_ref, be_ref, o_ref):
    dinv = _dinv_of(d_ref)
    t = (a_ref[0] + a_ref[1]) * dinv[:, None] + b_ref[...]
    t = t * (g_ref[...] * _BN_RS) + be_ref[...]
    t = jnp.maximum(t, 0.0)
    o_ref[...] = jnp.dot(t, w_ref[...],
                         preferred_element_type=_f32) * dinv[:, None]


def _tc_mid(acc, degp, w, b, g, be):
    return pl.pallas_call(
        _tc_mid_body,
        grid=(NP // _BR,),
        in_specs=[
            pl.BlockSpec((NC, _BR, H), lambda i: (0, i, 0)),
            pl.BlockSpec((NC, _BR), lambda i: (0, i)),
            pl.BlockSpec((H, H), lambda i: (0, 0)),
            pl.BlockSpec((1, H), lambda i: (0, 0)),
            pl.BlockSpec((1, H), lambda i: (0, 0)),
            pl.BlockSpec((1, H), lambda i: (0, 0)),
        ],
        out_specs=pl.BlockSpec((_BR, H), lambda i: (i, 0)),
        out_shape=jax.ShapeDtypeStruct((NP, H), _f32),
    )(acc, degp, w, b, g, be)


def _tc_post_body(a_ref, d_ref, b_ref, g_ref, be_ref, o_ref):
    dinv = _dinv_of(d_ref)
    t = (a_ref[0] + a_ref[1]) * dinv[:, None] + b_ref[...]
    o_ref[...] = t * (g_ref[...] * _BN_RS) + be_ref[...]


def _tc_post(acc, degp, b, g, be):
    return pl.pallas_call(
        _tc_post_body,
        grid=(NP // _BR,),
        in_specs=[
            pl.BlockSpec((NC, _BR, H), lambda i: (0, i, 0)),
            pl.BlockSpec((NC, _BR), lambda i: (0, i)),
            pl.BlockSpec((1, H), lambda i: (0, 0)),
            pl.BlockSpec((1, H), lambda i: (0, 0)),
            pl.BlockSpec((1, H), lambda i: (0, 0)),
        ],
        out_specs=pl.BlockSpec((_BR, H), lambda i: (i, 0)),
        out_shape=jax.ShapeDtypeStruct((NP, H), _f32),
    )(acc, degp, b, g, be)


def _tc_tail_body(p_ref, c_ref, a_ref, d1_ref, wl_ref, bl_ref, xp_ref, wa_ref,
                  wb_ref, w1_ref, b0_ref, g0_ref, be0_ref, b1_ref, g1_ref,
                  be1_ref, wlf_ref, blf_ref, nl_ref, gl_ref):
    deg1 = d1_ref[0]
    dinv1 = jnp.where(deg1 > 0, lax.rsqrt(deg1), 0.0)
    adj = (a_ref[0] + a_ref[1]) * dinv1[:, None] * dinv1[None, :]
    cnt = c_ref[0] + c_ref[1]
    pooled = (p_ref[0] + p_ref[1]) / jnp.maximum(cnt, 1.0)[:, None]
    nl = jnp.dot(pooled, wl_ref[...], preferred_element_type=_f32) + bl_ref[...]
    nl_ref[...] = nl
    h1 = jnp.dot(nl, wa_ref[...], preferred_element_type=_f32)
    h1 = h1 + xp_ref[0][:, None] * wb_ref[...]
    t = jnp.dot(adj, h1, preferred_element_type=_f32) + b0_ref[...]
    t = t * (g0_ref[...] * _BN_RS) + be0_ref[...]
    t = jnp.maximum(t, 0.0)
    h2 = jnp.dot(t, w1_ref[...], preferred_element_type=_f32)
    t = jnp.dot(adj, h2, preferred_element_type=_f32) + b1_ref[...]
    t = t * (g1_ref[...] * _BN_RS) + be1_ref[...]
    rows = lax.broadcasted_iota(_i32, (N1P, H), 0)
    t = jnp.where(rows < N1, t, 0.0)
    g_mean = jnp.sum(t, axis=0, keepdims=True) * (1.0 / N1)
    gl_ref[...] = jnp.dot(g_mean, wlf_ref[...],
                          preferred_element_type=_f32) + blf_ref[...]


def _tc_tail(pacc, cntp, adjp, deg1p, wl, bl, xp, wa, wb, w1, b0, g0, be0,
             b1, g1, be1, wlf, blf):
    return pl.pallas_call(
        _tc_tail_body,
        out_shape=(
            jax.ShapeDtypeStruct((N1P, H), _f32),
            jax.ShapeDtypeStruct((1, H), _f32),
        ),
    )(pacc, cntp, adjp, deg1p, wl, bl, xp, wa, wb, w1, b0, g0, be0, b1, g1,
      be1, wlf, blf)


def kernel(x, edge_index, pool1, x_pool1, edge_index_pool1,
           W0_p0, b0_p0, g0_p0, be0_p0, W1_p0, b1_p0, g1_p0, be1_p0,
           Wl_p0, bl_p0,
           W0_p1, b0_p1, g0_p1, be0_p1, W1_p1, b1_p1, g1_p1, be1_p1,
           Wl_p1, bl_p1):
    ei = edge_index.astype(_i32)
    pad_big = N + (jnp.arange(EP - E, dtype=_i32) % (NP - N))
    src_p = jnp.concatenate([ei[0], pad_big])
    dst_p = jnp.concatenate([ei[1], pad_big])
    x_p = jnp.pad(x, ((0, NP - N), (0, 0)))

    pool1_p = jnp.concatenate(
        [pool1.astype(_i32), jnp.full((NP - N,), N1, _i32)])
    ei1 = edge_index_pool1.astype(_i32)
    pad_sm = N1 + (jnp.arange(E1P - E1, dtype=_i32) % (N1P - N1))
    src1_p = jnp.concatenate([ei1[0], pad_sm])
    dst1_p = jnp.concatenate([ei1[1], pad_sm])

    dst_2d = dst_p.reshape(EP // 128, 128)
    pool1_2d = pool1_p.reshape(NP // 80, 80)
    dst1_2d = dst1_p.reshape(E1P // 80, 80)
    fidx1 = (dst1_p * N1P + src1_p).reshape(E1P // 80, 80)

    iota_np = jnp.arange(NP, dtype=_i32)
    xp_row = jnp.pad(x_pool1, ((0, N1P - N1), (0, 0))).reshape(1, N1P)

    wl0 = jnp.pad(Wl_p0, ((0, 0), (0, H - C)))
    bl0 = jnp.pad(bl_p0, (0, H - C)).reshape(1, H)
    wa = jnp.pad(W0_p1[:C], ((0, H - C), (0, 0)))
    wb = W0_p1[C:C + 1]
    wl1 = jnp.pad(Wl_p1, ((0, 0), (0, H - C)))
    bl1 = jnp.pad(bl_p1, (0, H - C)).reshape(1, H)

    r = lambda v: v.reshape(1, H)

    pre_k = _make_pre_kernel()
    scat_big = _make_row_scatter(NP, NP, EP, 80)
    scat_pool = _make_row_scatter(NP, N1P, NP, 80)

    degp, cntp, deg1p, adjp = pre_k(dst_2d, pool1_2d, dst1_2d, fidx1)
    degp = degp.reshape(NC, NP)
    adjp = adjp.reshape(NC, N1P, N1P)

    hs0 = _tc_in(x_p, W0_p0, degp)
    acc0 = scat_big(src_p, dst_p, hs0)
    hs1 = _tc_mid(acc0, degp, W1_p0, r(b0_p0), r(g0_p0), r(be0_p0))
    acc1 = scat_big(src_p, dst_p, hs1)
    hf = _tc_post(acc1, degp, r(b1_p0), r(g1_p0), r(be1_p0))

    pacc = scat_pool(iota_np, pool1_p, hf)
    nlf, gl = _tc_tail(pacc, cntp, adjp, deg1p, wl0, bl0, xp_row, wa, wb,
                       W1_p1, r(b0_p1), r(g0_p1), r(be0_p1),
                       r(b1_p1), r(g1_p1), r(be1_p1), wl1, bl1)

    return gl[:, :C], nlf[:N1, :C]
